# baseline (speedup 1.0000x reference)
import functools

import jax
import jax.numpy as jnp
import numpy as np
from jax import lax
from jax.experimental import pallas as pl
from jax.experimental.pallas import tpu as pltpu
from jax.experimental.pallas import tpu_sc as plsc

_B = 8
_N = 4096
_C = 768
_NPW = 8
_PATCH = 8
_W = 64
_NSLOT = 48
_NUM_KEEP = 44
_NOUT = 3072
_ROWS = _B * _NOUT
_NTILES = 32
_RPT = _ROWS // _NTILES
_CHUNK = 64
_NCHUNK = _RPT // _CHUNK


_NPATCH = 64
_H = 64


def _pool_matrix() -> np.ndarray:
    t = np.arange(_N)
    patch_id = (t // (_PATCH * _W)) * _NPW + (t % _W) // _PATCH
    return (patch_id[None, :] == np.arange(_NPATCH)[:, None]).astype(np.float32)


def _score_body(tok_ref, w1_ref, b1_ref, w2_ref, b2_ref, pt_ref, idx_ref):
    prec = jax.lax.Precision.DEFAULT
    tok = tok_ref[0]
    h = jnp.dot(tok, w1_ref[...], precision=prec,
                preferred_element_type=jnp.float32) + b1_ref[...]
    h = jax.nn.gelu(h)
    s = jnp.dot(h, w2_ref[...], precision=prec,
                preferred_element_type=jnp.float32) + b2_ref[...]
    s = jax.nn.sigmoid(s)
    imp_col = jnp.dot(pt_ref[...], s, precision=jax.lax.Precision.HIGHEST,
                      preferred_element_type=jnp.float32)

    ii = jax.lax.broadcasted_iota(jnp.int32, (_NPATCH, _NPATCH), 0)
    jj = jax.lax.broadcasted_iota(jnp.int32, (_NPATCH, _NPATCH), 1)
    eye = (ii == jj).astype(jnp.float32)
    imp_row = jnp.sum(eye * imp_col, axis=0, keepdims=True)

    gt = (imp_row > imp_col) | ((imp_row == imp_col) & (jj < ii))
    rank = jnp.sum(gt.astype(jnp.float32), axis=1, keepdims=True)
    keep = rank < float(_NUM_KEEP)
    ltri = (jj <= ii).astype(jnp.float32)
    pos = jnp.dot(ltri, keep.astype(jnp.float32),
                  preferred_element_type=jnp.float32) - 1.0

    kk = jax.lax.broadcasted_iota(jnp.int32, (_NPATCH, _NSLOT), 1).astype(jnp.float32)
    ids = jax.lax.broadcasted_iota(jnp.int32, (_NPATCH, _NSLOT), 0).astype(jnp.float32)
    onehot = keep & (pos == kk)
    top = jnp.sum(jnp.where(onehot, ids, 0.0), axis=0, keepdims=True)
    idx_ref[0] = top.astype(jnp.int32)


def _score_topk(tokens, W1, b1, W2, b2):
    Bsz, N, Ch = tokens.shape
    pt = jnp.asarray(_pool_matrix())
    idx = pl.pallas_call(
        _score_body,
        grid=(Bsz,),
        in_specs=[
            pl.BlockSpec((1, N, Ch), lambda b: (b, 0, 0)),
            pl.BlockSpec((Ch, W1.shape[1]), lambda b: (0, 0)),
            pl.BlockSpec((1, W1.shape[1]), lambda b: (0, 0)),
            pl.BlockSpec((W1.shape[1], 1), lambda b: (0, 0)),
            pl.BlockSpec((1, 1), lambda b: (0, 0)),
            pl.BlockSpec((_NPATCH, N), lambda b: (0, 0)),
        ],
        out_specs=pl.BlockSpec((1, 1, _NSLOT), lambda b: (b, 0, 0)),
        out_shape=jax.ShapeDtypeStruct((Bsz, 1, _NSLOT), jnp.int32),
    )(tokens, W1, b1.reshape(1, -1), W2, b2.reshape(1, 1), pt)
    return idx.reshape(Bsz, _NSLOT)


def _expand_indices(idx48):
    t = jnp.arange(_NOUT, dtype=jnp.int32)
    kr = t // (_PATCH * _W)
    i = (t // _W) % _PATCH
    kc = (t % _W) // _PATCH
    j = t % _PATCH
    slot = kr * _NPW + kc
    p = idx48[:, slot]
    src = (p // _NPW) * (_PATCH * _W) + (p % _NPW) * _PATCH + i * _W + j
    src = src + jnp.arange(_B, dtype=jnp.int32)[:, None] * _N
    return src.reshape(_ROWS)


def _sc_gather(tok_flat, src_idx):
    mesh = plsc.VectorSubcoreMesh(core_axis_name="c", subcore_axis_name="s")

    @functools.partial(
        pl.kernel,
        mesh=mesh,
        out_type=jax.ShapeDtypeStruct((_ROWS, _C), jnp.float32),
        scratch_types=[
            pltpu.VMEM((_RPT,), jnp.int32),
            pltpu.VMEM((_CHUNK, _C), jnp.float32),
            pltpu.VMEM((_CHUNK // 2, _C), jnp.float32),
            pltpu.SemaphoreType.DMA,
        ],
    )
    def k(tok_hbm, idx_hbm, out_hbm, idx_v, buf, zbuf, sem):
        wid = lax.axis_index("s") * 2 + lax.axis_index("c")
        base = wid * _RPT
        pltpu.sync_copy(idx_hbm.at[pl.ds(base, _RPT)], idx_v)
        is_pad_tile = (wid % 4) == 3

        @pl.when(jnp.logical_not(is_pad_tile))
        def _full():
            for c in range(_NCHUNK):
                pltpu.async_copy(
                    tok_hbm.at[idx_v.at[pl.ds(c * _CHUNK, _CHUNK)]],
                    buf, sem).wait()
                pltpu.sync_copy(buf,
                                out_hbm.at[pl.ds(base + c * _CHUNK, _CHUNK)])

        @pl.when(is_pad_tile)
        def _padded():
            def _zero_row(r, _):
                def _zero_lane(cc, __):
                    zbuf[r, pl.ds(cc * 16, 16)] = jnp.zeros((16,), jnp.float32)
                    return __
                return lax.fori_loop(0, _C // 16, _zero_lane, _)
            lax.fori_loop(0, _CHUNK // 2, _zero_row, 0)

            for c in range(4):
                pltpu.async_copy(
                    tok_hbm.at[idx_v.at[pl.ds(c * _CHUNK, _CHUNK)]],
                    buf, sem).wait()
                pltpu.sync_copy(buf,
                                out_hbm.at[pl.ds(base + c * _CHUNK, _CHUNK)])
            for c in range(4, _NCHUNK):
                half = _CHUNK // 2
                pltpu.async_copy(
                    tok_hbm.at[idx_v.at[pl.ds(c * _CHUNK, half)]],
                    buf.at[pl.ds(0, half)], sem).wait()
                pltpu.sync_copy(buf.at[pl.ds(0, half)],
                                out_hbm.at[pl.ds(base + c * _CHUNK, half)])
                pltpu.sync_copy(zbuf,
                                out_hbm.at[pl.ds(base + c * _CHUNK + half,
                                                 half)])

    return k(tok_flat, src_idx)


@jax.jit
def kernel(tokens, spatial_shape, W1, b1, W2, b2):
    Bsz, N, Ch = tokens.shape
    idx48 = _score_topk(tokens, W1, b1, W2, b2)
    src = _expand_indices(idx48)
    tok_flat = tokens.reshape(Bsz * N, Ch)
    out = _sc_gather(tok_flat, src)
    return out.reshape(Bsz, _NOUT, Ch)


# speedup vs baseline: 2.0085x; 2.0085x over previous
import jax
import jax.numpy as jnp
import numpy as np
from jax.experimental import pallas as pl
from jax.experimental.pallas import tpu as pltpu

_B = 8
_H = 64
_W = 64
_C = 768
_PATCH = 8
_NPH = _H // _PATCH
_NPW = _W // _PATCH
_NPATCH = _NPH * _NPW
_N = _H * _W
_NUM_KEEP = max(10, int(_NPATCH * 0.7))
_NKH = (_NUM_KEEP + _NPW - 1) // _NPW
_NKW = min(_NUM_KEEP, _NPW)
_NSLOT = _NKH * _NKW
_NOUT = _NKH * _PATCH * _W


def _pool_matrix() -> np.ndarray:
    t = np.arange(_N)
    patch_id = (t // (_PATCH * _W)) * _NPW + (t % _W) // _PATCH
    return (patch_id[None, :] == np.arange(_NPATCH)[:, None]).astype(np.float32)


def _topk_slots(imp_col):
    ii = jax.lax.broadcasted_iota(jnp.int32, (_NPATCH, _NPATCH), 0)
    jj = jax.lax.broadcasted_iota(jnp.int32, (_NPATCH, _NPATCH), 1)
    eye = (ii == jj).astype(jnp.float32)
    imp_row = jnp.sum(eye * imp_col, axis=0, keepdims=True)

    gt = (imp_row > imp_col) | ((imp_row == imp_col) & (jj < ii))
    rank = jnp.sum(gt.astype(jnp.float32), axis=1, keepdims=True)
    keep = rank < float(_NUM_KEEP)
    ltri = (jj <= ii).astype(jnp.float32)
    pos = jnp.dot(ltri, keep.astype(jnp.float32),
                  preferred_element_type=jnp.float32) - 1.0

    kk = jax.lax.broadcasted_iota(jnp.int32, (_NPATCH, _NSLOT), 1)
    ids = jax.lax.broadcasted_iota(jnp.int32, (_NPATCH, _NSLOT), 0)
    onehot = keep & (pos == kk.astype(jnp.float32))
    return jnp.sum(jnp.where(onehot, ids.astype(jnp.float32), 0.0),
                   axis=0, keepdims=True)


def _fused_body(tok_ref, w1_ref, b1_ref, w2_ref, b2_ref, pt_ref, out_ref,
                idxv_ref, idxs_ref, sem):
    prec = jax.lax.Precision.DEFAULT
    tok = tok_ref[0]
    h = jnp.dot(tok, w1_ref[...], precision=prec,
                preferred_element_type=jnp.float32) + b1_ref[...]
    h = jax.nn.gelu(h)
    s = jnp.dot(h, w2_ref[...], precision=prec,
                preferred_element_type=jnp.float32) + b2_ref[...]
    s = jax.nn.sigmoid(s)
    imp_col = jnp.dot(pt_ref[...], s, precision=jax.lax.Precision.HIGHEST,
                      preferred_element_type=jnp.float32)

    idxv_ref[...] = _topk_slots(imp_col).astype(jnp.int32)
    copy = pltpu.make_async_copy(idxv_ref, idxs_ref, sem)
    copy.start()

    pad_w = (_NKW - (_NUM_KEEP - (_NKH - 1) * _NKW)) * _PATCH
    for i in range(_PATCH):
        r0 = ((_NKH - 1) * _PATCH + i) * _W + (_W - pad_w)
        out_ref[0, r0:r0 + pad_w, :] = jnp.zeros((pad_w, _C), jnp.float32)

    copy.wait()

    for slot in range(_NUM_KEEP):
        kr, kc = divmod(slot, _NKW)
        p = idxs_ref[0, slot]
        src = (p // _NPW) * (_PATCH * _W) + (p % _NPW) * _PATCH
        for i in range(_PATCH):
            r0 = (kr * _PATCH + i) * _W + kc * _PATCH
            out_ref[0, r0:r0 + _PATCH, :] = \
                tok_ref[0, pl.ds(src + i * _W, _PATCH), :]


@jax.jit
def kernel(tokens, spatial_shape, W1, b1, W2, b2):
    Bsz, N, Ch = tokens.shape
    pt = jnp.asarray(_pool_matrix())

    out = pl.pallas_call(
        _fused_body,
        grid=(Bsz,),
        in_specs=[
            pl.BlockSpec((1, N, Ch), lambda b: (b, 0, 0)),
            pl.BlockSpec((Ch, W1.shape[1]), lambda b: (0, 0)),
            pl.BlockSpec((1, W1.shape[1]), lambda b: (0, 0)),
            pl.BlockSpec((W1.shape[1], 1), lambda b: (0, 0)),
            pl.BlockSpec((1, 1), lambda b: (0, 0)),
            pl.BlockSpec((_NPATCH, N), lambda b: (0, 0)),
        ],
        out_specs=pl.BlockSpec((1, _NOUT, Ch), lambda b: (b, 0, 0)),
        out_shape=jax.ShapeDtypeStruct((Bsz, _NOUT, Ch), jnp.float32),
        scratch_shapes=[
            pltpu.VMEM((1, _NSLOT), jnp.int32),
            pltpu.SMEM((1, _NSLOT), jnp.int32),
            pltpu.SemaphoreType.DMA,
        ],
    )(tokens, W1, b1.reshape(1, -1), W2, b2.reshape(1, 1), pt)

    return out
